# Initial kernel scaffold; baseline (speedup 1.0000x reference)
import functools

import jax
import jax.numpy as jnp
from jax import lax
from jax.experimental import pallas as pl
from jax.experimental.pallas import tpu as pltpu
from jax.experimental.pallas import tpu_sc as plsc

N = 10000
E = 320000
C = 128

NC = 2
NS = 16
NW = NC * NS
CHUNK = 128
K = -(-E // (NW * CHUNK))
E_PAD = NW * K * CHUNK
N_ACC = 10048
ROWS_PER_TILE = N_ACC // NS
DEG_W = 16

_MESH = plsc.VectorSubcoreMesh(
    core_axis_name="c", subcore_axis_name="s", num_cores=NC, num_subcores=NS
)


@functools.partial(
    pl.kernel,
    out_type=jax.ShapeDtypeStruct((NC, N_ACC, DEG_W), jnp.float32),
    mesh=_MESH,
    scratch_types=[
        pltpu.VMEM((K, CHUNK), jnp.int32),
        pltpu.VMEM((CHUNK, DEG_W), jnp.float32),
        pltpu.VMEM_SHARED((N_ACC, DEG_W), jnp.float32),
    ],
)
def _deg_kernel(dst_hbm, ones_hbm, zeros_hbm, out_hbm, dst_v, ones_v, acc):
    cid = lax.axis_index("c")
    sid = lax.axis_index("s")
    wid = sid * NC + cid
    r0 = sid * ROWS_PER_TILE
    pltpu.sync_copy(dst_hbm.at[wid], dst_v)
    pltpu.sync_copy(ones_hbm, ones_v)
    pltpu.sync_copy(
        zeros_hbm.at[pl.ds(r0, ROWS_PER_TILE)], acc.at[pl.ds(r0, ROWS_PER_TILE)]
    )
    plsc.subcore_barrier()

    def body(j, carry):
        pltpu.sync_copy(ones_v, acc.at[dst_v.at[j]], add=True)
        return carry

    lax.fori_loop(0, K, body, 0)
    plsc.subcore_barrier()
    pltpu.sync_copy(
        acc.at[pl.ds(r0, ROWS_PER_TILE)], out_hbm.at[cid, pl.ds(r0, ROWS_PER_TILE)]
    )


@functools.partial(
    pl.kernel,
    out_type=jax.ShapeDtypeStruct((NC, N_ACC, C), jnp.float32),
    mesh=_MESH,
    scratch_types=[
        pltpu.VMEM((K, CHUNK), jnp.int32),
        pltpu.VMEM((K, CHUNK), jnp.int32),
        pltpu.VMEM((CHUNK, C), jnp.float32),
        pltpu.VMEM_SHARED((N_ACC, C), jnp.float32),
        pltpu.SemaphoreType.DMA,
    ],
)
def _agg_kernel(y_hbm, src_hbm, dst_hbm, zeros_hbm, out_hbm,
                src_v, dst_v, rows_v, acc, sem):
    cid = lax.axis_index("c")
    sid = lax.axis_index("s")
    wid = sid * NC + cid
    r0 = sid * ROWS_PER_TILE
    pltpu.sync_copy(src_hbm.at[wid], src_v)
    pltpu.sync_copy(dst_hbm.at[wid], dst_v)
    pltpu.sync_copy(
        zeros_hbm.at[pl.ds(r0, ROWS_PER_TILE)], acc.at[pl.ds(r0, ROWS_PER_TILE)]
    )
    plsc.subcore_barrier()

    def body(j, carry):
        pltpu.async_copy(y_hbm.at[src_v.at[j]], rows_v, sem).wait()
        pltpu.sync_copy(rows_v, acc.at[dst_v.at[j]], add=True)
        return carry

    lax.fori_loop(0, K, body, 0)
    plsc.subcore_barrier()
    pltpu.sync_copy(
        acc.at[pl.ds(r0, ROWS_PER_TILE)], out_hbm.at[cid, pl.ds(r0, ROWS_PER_TILE)]
    )


_BLK = 1000


def _dinv_of(degp_blk):
    deg = degp_blk[0, :, 0] + degp_blk[1, :, 0] + 1.0
    return lax.rsqrt(deg)


def _y1_body(x_ref, w_ref, degp_ref, y_ref):
    dinv = _dinv_of(degp_ref[...])
    y_ref[...] = jnp.dot(
        x_ref[...], w_ref[...], preferred_element_type=jnp.float32
    ) * dinv[:, None]


def _mid_body(aggp_ref, y_ref, degp_ref, b_ref, w_ref, y2_ref):
    dinv = _dinv_of(degp_ref[...])
    s = aggp_ref[0] + aggp_ref[1] + y_ref[...]
    h = jnp.maximum(dinv[:, None] * s + b_ref[...], 0.0)
    y2_ref[...] = jnp.dot(
        h, w_ref[...], preferred_element_type=jnp.float32
    ) * dinv[:, None]


def _fin_body(aggp_ref, y_ref, degp_ref, b_ref, w_ref, fcb_ref, out_ref):
    dinv = _dinv_of(degp_ref[...])
    s = aggp_ref[0] + aggp_ref[1] + y_ref[...]
    h = dinv[:, None] * s + b_ref[...]
    out_ref[...] = jnp.dot(
        h, w_ref[...], preferred_element_type=jnp.float32
    ) + fcb_ref[...]


def _row_spec(w):
    return pl.BlockSpec((_BLK, w), lambda i: (i, 0))


_DEG_SPEC = pl.BlockSpec((NC, _BLK, DEG_W), lambda i: (0, i, 0))
_AGG_SPEC = pl.BlockSpec((NC, _BLK, C), lambda i: (0, i, 0))
_W_SPEC = pl.BlockSpec((C, C), lambda i: (0, 0))
_B_SPEC = pl.BlockSpec((1, C), lambda i: (0, 0))


def _tc_y1(x, w1, degp):
    return pl.pallas_call(
        _y1_body,
        grid=(N // _BLK,),
        in_specs=[_row_spec(C), _W_SPEC, _DEG_SPEC],
        out_specs=_row_spec(C),
        out_shape=jax.ShapeDtypeStruct((N, C), jnp.float32),
    )(x, w1, degp)


def _tc_mid(aggp, y1, degp, b1, w2):
    return pl.pallas_call(
        _mid_body,
        grid=(N // _BLK,),
        in_specs=[_AGG_SPEC, _row_spec(C), _DEG_SPEC, _B_SPEC, _W_SPEC],
        out_specs=_row_spec(C),
        out_shape=jax.ShapeDtypeStruct((N, C), jnp.float32),
    )(aggp, y1, degp, b1, w2)


def _tc_fin(aggp, y2, degp, b2, fcw, fcb):
    return pl.pallas_call(
        _fin_body,
        grid=(N // _BLK,),
        in_specs=[_AGG_SPEC, _row_spec(C), _DEG_SPEC, _B_SPEC, _W_SPEC, _B_SPEC],
        out_specs=_row_spec(C),
        out_shape=jax.ShapeDtypeStruct((N, C), jnp.float32),
    )(aggp, y2, degp, b2, fcw, fcb)


@jax.jit
def _run(x, edge_index, W1, b1, W2, b2, fcW, fcb):
    src = edge_index[0].astype(jnp.int32)
    dst = edge_index[1].astype(jnp.int32)
    pad = E_PAD - E
    src_t = jnp.concatenate([src, jnp.zeros((pad,), jnp.int32)]).reshape(
        NW, K, CHUNK
    )
    dst_t = jnp.concatenate([dst, jnp.full((pad,), N, jnp.int32)]).reshape(
        NW, K, CHUNK
    )
    ones16 = jnp.ones((CHUNK, DEG_W), jnp.float32)
    zeros16 = jnp.zeros((N_ACC, DEG_W), jnp.float32)
    zerosC = jnp.zeros((N_ACC, C), jnp.float32)

    degp = _deg_kernel(dst_t, ones16, zeros16)[:, :N, :]

    y1 = _tc_y1(x, W1, degp)
    agg1 = _agg_kernel(y1, src_t, dst_t, zerosC)[:, :N, :]

    y2 = _tc_mid(agg1, y1, degp, b1.reshape(1, C), W2)
    agg2 = _agg_kernel(y2, src_t, dst_t, zerosC)[:, :N, :]

    fcw_p = jnp.zeros((C, C), jnp.float32).at[:, : fcW.shape[1]].set(fcW)
    fcb_p = jnp.zeros((1, C), jnp.float32).at[0, : fcb.shape[0]].set(fcb)
    out = _tc_fin(agg2, y2, degp, b2.reshape(1, C), fcw_p, fcb_p)
    return out[:, : fcW.shape[1]]


def kernel(x, edge_index, W1, b1, W2, b2, fcW, fcb):
    return _run(x, edge_index, W1, b1, W2, b2, fcW, fcb)


# speedup vs baseline: 12.2399x; 12.2399x over previous
import functools

import jax
import jax.numpy as jnp
from jax import lax
from jax.experimental import pallas as pl
from jax.experimental.pallas import tpu as pltpu
from jax.experimental.pallas import tpu_sc as plsc

N = 10000
E = 320000
C = 128

NC = 2
NS = 16
NW = NC * NS
CHUNK = 128
K = -(-E // (NW * CHUNK))
E_PAD = NW * K * CHUNK
N_ACC = 10112
ROWS_PER_TILE = N_ACC // NS
DEG_W = 16

_MESH = plsc.VectorSubcoreMesh(
    core_axis_name="c", subcore_axis_name="s", num_cores=NC, num_subcores=NS
)


EP = K * CHUNK
HR = 128
N_DEG = HR * CHUNK


@functools.partial(
    pl.kernel,
    out_type=jax.ShapeDtypeStruct((NC, N_DEG, CHUNK), jnp.float32),
    mesh=_MESH,
    scratch_types=[
        pltpu.VMEM((EP,), jnp.int32),
        pltpu.VMEM((HR, CHUNK), jnp.float32),
        pltpu.VMEM((1, CHUNK), jnp.int32),
        pltpu.VMEM((8, CHUNK), jnp.float32),
        pltpu.VMEM((512, CHUNK), jnp.float32),
        pltpu.VMEM_SHARED((HR, CHUNK), jnp.float32),
    ],
    compiler_params=pltpu.CompilerParams(needs_layout_passes=False),
)
def _deg_kernel(dst_hbm, zeros_hbm, out_hbm, dst_v, hist, ident, conv, outb, acc):
    cid = lax.axis_index("c")
    sid = lax.axis_index("s")
    wid = sid * NC + cid
    pltpu.sync_copy(dst_hbm.at[wid], dst_v)
    pltpu.sync_copy(zeros_hbm, hist)
    r8 = pl.multiple_of(sid * 8, 8)
    pltpu.sync_copy(zeros_hbm.at[pl.ds(r8, 8)], acc.at[pl.ds(r8, 8)])
    lanes = lax.iota(jnp.int32, 16)
    for c in range(8):
        ident[0, pl.ds(c * 16, 16)] = lanes + c * 16

    ones = jnp.full((16,), 1.0, jnp.float32)

    def body(i, carry):
        base = i * CHUNK
        for u in range(CHUNK // 16):
            idx = dst_v[pl.ds(base + u * 16, 16)]
            plsc.addupdate_scatter(hist, [idx >> 7, idx & 127], ones)
        return carry

    lax.fori_loop(0, K, body, 0)
    plsc.subcore_barrier()
    pltpu.sync_copy(hist, acc.at[ident.at[0]], add=True)
    plsc.subcore_barrier()
    pltpu.sync_copy(acc.at[pl.ds(r8, 8)], conv)
    zeros16i = jnp.zeros((16,), jnp.int32)
    for p in range(2):
        for j in range(32):
            g = p * 32 + j
            vals = conv[g // 8, pl.ds((g % 8) * 16, 16)]
            plsc.store_scatter(outb, [lanes + j * 16, zeros16i], vals)
        pltpu.sync_copy(
            outb, out_hbm.at[cid, pl.ds(sid * 1024 + p * 512, 512)]
        )


@functools.partial(
    pl.kernel,
    out_type=jax.ShapeDtypeStruct((NC, N_ACC, C), jnp.float32),
    mesh=_MESH,
    scratch_types=[
        pltpu.VMEM((K, CHUNK), jnp.int32),
        pltpu.VMEM((K, CHUNK), jnp.int32),
        pltpu.VMEM((CHUNK, C), jnp.float32),
        pltpu.VMEM_SHARED((N_ACC, C), jnp.float32),
        pltpu.SemaphoreType.DMA,
    ],
)
def _agg_kernel(y_hbm, src_hbm, dst_hbm, zeros_hbm, out_hbm,
                src_v, dst_v, rows_v, acc, sem):
    cid = lax.axis_index("c")
    sid = lax.axis_index("s")
    wid = sid * NC + cid
    r0 = pl.multiple_of(sid * ROWS_PER_TILE, 8)
    pltpu.sync_copy(src_hbm.at[wid], src_v)
    pltpu.sync_copy(dst_hbm.at[wid], dst_v)
    pltpu.sync_copy(
        zeros_hbm.at[pl.ds(r0, ROWS_PER_TILE)], acc.at[pl.ds(r0, ROWS_PER_TILE)]
    )
    plsc.subcore_barrier()

    def body(j, carry):
        pltpu.async_copy(y_hbm.at[src_v.at[j]], rows_v, sem).wait()
        pltpu.sync_copy(rows_v, acc.at[dst_v.at[j]], add=True)
        return carry

    lax.fori_loop(0, K, body, 0)
    plsc.subcore_barrier()
    pltpu.sync_copy(
        acc.at[pl.ds(r0, ROWS_PER_TILE)], out_hbm.at[cid, pl.ds(r0, ROWS_PER_TILE)]
    )


_BLK = 1000


def _dinv_of(degp_blk):
    deg = degp_blk[0, :, 0] + degp_blk[1, :, 0] + 1.0
    return lax.rsqrt(deg)


def _y1_body(x_ref, w_ref, degp_ref, y_ref):
    dinv = _dinv_of(degp_ref[...])
    y_ref[...] = jnp.dot(
        x_ref[...], w_ref[...], preferred_element_type=jnp.float32
    ) * dinv[:, None]


def _mid_body(aggp_ref, y_ref, degp_ref, b_ref, w_ref, y2_ref):
    dinv = _dinv_of(degp_ref[...])
    s = aggp_ref[0] + aggp_ref[1] + y_ref[...]
    h = jnp.maximum(dinv[:, None] * s + b_ref[...], 0.0)
    y2_ref[...] = jnp.dot(
        h, w_ref[...], preferred_element_type=jnp.float32
    ) * dinv[:, None]


def _fin_body(aggp_ref, y_ref, degp_ref, b_ref, w_ref, fcb_ref, out_ref):
    dinv = _dinv_of(degp_ref[...])
    s = aggp_ref[0] + aggp_ref[1] + y_ref[...]
    h = dinv[:, None] * s + b_ref[...]
    out_ref[...] = jnp.dot(
        h, w_ref[...], preferred_element_type=jnp.float32
    ) + fcb_ref[...]


def _row_spec(w):
    return pl.BlockSpec((_BLK, w), lambda i: (i, 0))


_DEG_SPEC = pl.BlockSpec((NC, _BLK, DEG_W), lambda i: (0, i, 0))
_AGG_SPEC = pl.BlockSpec((NC, _BLK, C), lambda i: (0, i, 0))
_W_SPEC = pl.BlockSpec((C, C), lambda i: (0, 0))
_B_SPEC = pl.BlockSpec((1, C), lambda i: (0, 0))


def _tc_y1(x, w1, degp):
    return pl.pallas_call(
        _y1_body,
        grid=(N // _BLK,),
        in_specs=[_row_spec(C), _W_SPEC, _DEG_SPEC],
        out_specs=_row_spec(C),
        out_shape=jax.ShapeDtypeStruct((N, C), jnp.float32),
    )(x, w1, degp)


def _tc_mid(aggp, y1, degp, b1, w2):
    return pl.pallas_call(
        _mid_body,
        grid=(N // _BLK,),
        in_specs=[_AGG_SPEC, _row_spec(C), _DEG_SPEC, _B_SPEC, _W_SPEC],
        out_specs=_row_spec(C),
        out_shape=jax.ShapeDtypeStruct((N, C), jnp.float32),
    )(aggp, y1, degp, b1, w2)


def _tc_fin(aggp, y2, degp, b2, fcw, fcb):
    return pl.pallas_call(
        _fin_body,
        grid=(N // _BLK,),
        in_specs=[_AGG_SPEC, _row_spec(C), _DEG_SPEC, _B_SPEC, _W_SPEC, _B_SPEC],
        out_specs=_row_spec(C),
        out_shape=jax.ShapeDtypeStruct((N, C), jnp.float32),
    )(aggp, y2, degp, b2, fcw, fcb)


@jax.jit
def _run(x, edge_index, W1, b1, W2, b2, fcW, fcb):
    src = edge_index[0].astype(jnp.int32)
    dst = edge_index[1].astype(jnp.int32)
    pad = E_PAD - E
    src_t = jnp.concatenate([src, jnp.zeros((pad,), jnp.int32)]).reshape(
        NW, K, CHUNK
    )
    dst_t = jnp.concatenate([dst, jnp.full((pad,), N, jnp.int32)]).reshape(
        NW, K, CHUNK
    )
    dst_flat = dst_t.reshape(NW, EP)
    zerosHR = jnp.zeros((HR, CHUNK), jnp.float32)
    zerosC = jnp.zeros((N_ACC, C), jnp.float32)

    degp = _deg_kernel(dst_flat, zerosHR)[:, :N, :DEG_W]

    y1 = _tc_y1(x, W1, degp)
    agg1 = _agg_kernel(y1, src_t, dst_t, zerosC)[:, :N, :]

    y2 = _tc_mid(agg1, y1, degp, b1.reshape(1, C), W2)
    agg2 = _agg_kernel(y2, src_t, dst_t, zerosC)[:, :N, :]

    fcw_p = jnp.zeros((C, C), jnp.float32).at[:, : fcW.shape[1]].set(fcW)
    fcb_p = jnp.zeros((1, C), jnp.float32).at[0, : fcb.shape[0]].set(fcb)
    out = _tc_fin(agg2, y2, degp, b2.reshape(1, C), fcw_p, fcb_p)
    return out[:, : fcW.shape[1]]


def kernel(x, edge_index, W1, b1, W2, b2, fcW, fcb):
    return _run(x, edge_index, W1, b1, W2, b2, fcW, fcb)


# speedup vs baseline: 26.2546x; 2.1450x over previous
import functools

import jax
import jax.numpy as jnp
from jax import lax
from jax.experimental import pallas as pl
from jax.experimental.pallas import tpu as pltpu
from jax.experimental.pallas import tpu_sc as plsc

N = 10000
E = 320000
C = 128

NC = 2
NS = 16
NW = NC * NS
CHUNK = 128
GK = 10
K = GK * 8
E_PAD = NW * K * CHUNK
N_ACC = 10112
ROWS_PER_TILE = N_ACC // NS
DEG_W = 16

_MESH = plsc.VectorSubcoreMesh(
    core_axis_name="c", subcore_axis_name="s", num_cores=NC, num_subcores=NS
)


EP = K * CHUNK
HR = 128
N_DEG = HR * CHUNK


@functools.partial(
    pl.kernel,
    out_type=jax.ShapeDtypeStruct((NC, N_DEG, CHUNK), jnp.float32),
    mesh=_MESH,
    scratch_types=[
        pltpu.VMEM((EP,), jnp.int32),
        pltpu.VMEM((HR, CHUNK), jnp.float32),
        pltpu.VMEM((1, CHUNK), jnp.int32),
        pltpu.VMEM((8, CHUNK), jnp.float32),
        pltpu.VMEM((512, CHUNK), jnp.float32),
        pltpu.VMEM_SHARED((HR, CHUNK), jnp.float32),
    ],
    compiler_params=pltpu.CompilerParams(needs_layout_passes=False),
)
def _deg_kernel(dst_hbm, zeros_hbm, out_hbm, dst_v, hist, ident, conv, outb, acc):
    cid = lax.axis_index("c")
    sid = lax.axis_index("s")
    wid = sid * NC + cid
    pltpu.sync_copy(dst_hbm.at[wid], dst_v)
    pltpu.sync_copy(zeros_hbm, hist)
    r8 = pl.multiple_of(sid * 8, 8)
    pltpu.sync_copy(zeros_hbm.at[pl.ds(r8, 8)], acc.at[pl.ds(r8, 8)])
    lanes = lax.iota(jnp.int32, 16)
    for c in range(8):
        ident[0, pl.ds(c * 16, 16)] = lanes + c * 16

    ones = jnp.full((16,), 1.0, jnp.float32)

    def body(i, carry):
        base = i * CHUNK
        for u in range(CHUNK // 16):
            idx = dst_v[pl.ds(base + u * 16, 16)]
            plsc.addupdate_scatter(hist, [idx >> 7, idx & 127], ones)
        return carry

    lax.fori_loop(0, K, body, 0)
    plsc.subcore_barrier()
    pltpu.sync_copy(hist, acc.at[ident.at[0]], add=True)
    plsc.subcore_barrier()
    pltpu.sync_copy(acc.at[pl.ds(r8, 8)], conv)
    zeros16i = jnp.zeros((16,), jnp.int32)
    for p in range(2):
        for j in range(32):
            g = p * 32 + j
            vals = conv[g // 8, pl.ds((g % 8) * 16, 16)]
            plsc.store_scatter(outb, [lanes + j * 16, zeros16i], vals)
        pltpu.sync_copy(
            outb, out_hbm.at[cid, pl.ds(sid * 1024 + p * 512, 512)]
        )


@functools.partial(
    pl.kernel,
    out_type=jax.ShapeDtypeStruct((NC, N_ACC, C), jnp.float32),
    mesh=_MESH,
    scratch_types=[
        pltpu.VMEM((3, 8, CHUNK), jnp.int32),
        pltpu.VMEM((3, 8, CHUNK), jnp.int32),
        pltpu.VMEM((2, CHUNK, C), jnp.float32),
        pltpu.VMEM_SHARED((N_ACC, C), jnp.float32),
        pltpu.SemaphoreType.DMA,
        pltpu.SemaphoreType.DMA,
        pltpu.SemaphoreType.DMA,
    ],
)
def _agg_kernel(y_hbm, src_hbm, dst_hbm, zeros_hbm, out_hbm,
                sgrp, dgrp, rows, acc, gsem, ssem, isem):
    cid = lax.axis_index("c")
    sid = lax.axis_index("s")
    wid = sid * NC + cid
    r0 = pl.multiple_of(sid * ROWS_PER_TILE, 8)
    pltpu.sync_copy(
        zeros_hbm.at[pl.ds(r0, ROWS_PER_TILE)], acc.at[pl.ds(r0, ROWS_PER_TILE)]
    )
    pltpu.sync_copy(src_hbm.at[wid, pl.ds(0, 8)], sgrp.at[0])
    pltpu.sync_copy(dst_hbm.at[wid, pl.ds(0, 8)], dgrp.at[0])
    pltpu.async_copy(src_hbm.at[wid, pl.ds(8, 8)], sgrp.at[1], isem)
    pltpu.async_copy(dst_hbm.at[wid, pl.ds(8, 8)], dgrp.at[1], isem)
    plsc.subcore_barrier()
    pltpu.async_copy(y_hbm.at[sgrp.at[0, 0]], rows.at[0], gsem)

    def body(j, carry):
        cur = j % 2
        g = j // 8
        r = j % 8
        slot = g % 3
        pltpu.make_async_copy(
            y_hbm.at[sgrp.at[slot, r]], rows.at[cur], gsem
        ).wait()
        @pl.when(j >= 1)
        def _():
            pltpu.make_async_copy(
                rows.at[1 - cur], acc.at[dgrp.at[slot, r]], ssem
            ).wait()

        pltpu.async_copy(rows.at[cur], acc.at[dgrp.at[slot, r]], ssem, add=True)
        @pl.when(jnp.logical_and(r == 7, g + 2 < GK))
        def _():
            s2 = (g + 2) % 3
            o2 = pl.multiple_of((g + 2) * 8, 8)
            pltpu.async_copy(src_hbm.at[wid, pl.ds(o2, 8)], sgrp.at[s2], isem)
            pltpu.async_copy(dst_hbm.at[wid, pl.ds(o2, 8)], dgrp.at[s2], isem)

        @pl.when(j + 1 < K)
        def _():
            g1 = (j + 1) // 8
            r1 = (j + 1) % 8
            s1 = g1 % 3
            @pl.when(r1 == 0)
            def _():
                o1 = pl.multiple_of(g1 * 8, 8)
                pltpu.make_async_copy(
                    src_hbm.at[wid, pl.ds(o1, 8)], sgrp.at[s1], isem
                ).wait()
                pltpu.make_async_copy(
                    dst_hbm.at[wid, pl.ds(o1, 8)], dgrp.at[s1], isem
                ).wait()

            pltpu.async_copy(y_hbm.at[sgrp.at[s1, r1]], rows.at[1 - cur], gsem)

        return carry

    lax.fori_loop(0, K, body, 0)
    pltpu.make_async_copy(rows.at[0], acc.at[dgrp.at[0, 0]], ssem).wait()
    plsc.subcore_barrier()
    pltpu.sync_copy(
        acc.at[pl.ds(r0, ROWS_PER_TILE)], out_hbm.at[cid, pl.ds(r0, ROWS_PER_TILE)]
    )


_BLK = 1000


def _dinv_of(degp_blk):
    deg = degp_blk[0, :, 0] + degp_blk[1, :, 0] + 1.0
    return lax.rsqrt(deg)


def _y1_body(x_ref, w_ref, degp_ref, y_ref):
    dinv = _dinv_of(degp_ref[...])
    y_ref[...] = jnp.dot(
        x_ref[...], w_ref[...], preferred_element_type=jnp.float32
    ) * dinv[:, None]


def _mid_body(aggp_ref, y_ref, degp_ref, b_ref, w_ref, y2_ref):
    dinv = _dinv_of(degp_ref[...])
    s = aggp_ref[0] + aggp_ref[1] + y_ref[...]
    h = jnp.maximum(dinv[:, None] * s + b_ref[...], 0.0)
    y2_ref[...] = jnp.dot(
        h, w_ref[...], preferred_element_type=jnp.float32
    ) * dinv[:, None]


def _fin_body(aggp_ref, y_ref, degp_ref, b_ref, w_ref, fcb_ref, out_ref):
    dinv = _dinv_of(degp_ref[...])
    s = aggp_ref[0] + aggp_ref[1] + y_ref[...]
    h = dinv[:, None] * s + b_ref[...]
    out_ref[...] = jnp.dot(
        h, w_ref[...], preferred_element_type=jnp.float32
    ) + fcb_ref[...]


def _row_spec(w):
    return pl.BlockSpec((_BLK, w), lambda i: (i, 0))


_DEG_SPEC = pl.BlockSpec((NC, _BLK, DEG_W), lambda i: (0, i, 0))
_AGG_SPEC = pl.BlockSpec((NC, _BLK, C), lambda i: (0, i, 0))
_W_SPEC = pl.BlockSpec((C, C), lambda i: (0, 0))
_B_SPEC = pl.BlockSpec((1, C), lambda i: (0, 0))


def _tc_y1(x, w1, degp):
    return pl.pallas_call(
        _y1_body,
        grid=(N // _BLK,),
        in_specs=[_row_spec(C), _W_SPEC, _DEG_SPEC],
        out_specs=_row_spec(C),
        out_shape=jax.ShapeDtypeStruct((N, C), jnp.float32),
    )(x, w1, degp)


def _tc_mid(aggp, y1, degp, b1, w2):
    return pl.pallas_call(
        _mid_body,
        grid=(N // _BLK,),
        in_specs=[_AGG_SPEC, _row_spec(C), _DEG_SPEC, _B_SPEC, _W_SPEC],
        out_specs=_row_spec(C),
        out_shape=jax.ShapeDtypeStruct((N, C), jnp.float32),
    )(aggp, y1, degp, b1, w2)


def _tc_fin(aggp, y2, degp, b2, fcw, fcb):
    return pl.pallas_call(
        _fin_body,
        grid=(N // _BLK,),
        in_specs=[_AGG_SPEC, _row_spec(C), _DEG_SPEC, _B_SPEC, _W_SPEC, _B_SPEC],
        out_specs=_row_spec(C),
        out_shape=jax.ShapeDtypeStruct((N, C), jnp.float32),
    )(aggp, y2, degp, b2, fcw, fcb)


@jax.jit
def _run(x, edge_index, W1, b1, W2, b2, fcW, fcb):
    src = edge_index[0].astype(jnp.int32)
    dst = edge_index[1].astype(jnp.int32)
    pad = E_PAD - E
    pad_ids = jnp.arange(pad, dtype=jnp.int32)
    src_t = jnp.concatenate([src, pad_ids % 128]).reshape(NW, K, CHUNK)
    dst_t = jnp.concatenate([dst, N + pad_ids % (N_ACC - N)]).reshape(
        NW, K, CHUNK
    )
    dst_flat = dst_t.reshape(NW, EP)
    zerosHR = jnp.zeros((HR, CHUNK), jnp.float32)
    zerosC = jnp.zeros((N_ACC, C), jnp.float32)

    degp = _deg_kernel(dst_flat, zerosHR)[:, :N, :DEG_W]

    y1 = _tc_y1(x, W1, degp)
    agg1 = _agg_kernel(y1, src_t, dst_t, zerosC)[:, :N, :]

    y2 = _tc_mid(agg1, y1, degp, b1.reshape(1, C), W2)
    agg2 = _agg_kernel(y2, src_t, dst_t, zerosC)[:, :N, :]

    fcw_p = jnp.zeros((C, C), jnp.float32).at[:, : fcW.shape[1]].set(fcW)
    fcb_p = jnp.zeros((1, C), jnp.float32).at[0, : fcb.shape[0]].set(fcb)
    out = _tc_fin(agg2, y2, degp, b2.reshape(1, C), fcw_p, fcb_p)
    return out[:, : fcW.shape[1]]


def kernel(x, edge_index, W1, b1, W2, b2, fcW, fcb):
    return _run(x, edge_index, W1, b1, W2, b2, fcW, fcb)
